# baseline (speedup 1.0000x reference)
import functools

import jax
import jax.numpy as jnp
from jax import lax
from jax.experimental import pallas as pl
from jax.experimental.pallas import tpu as pltpu
from jax.experimental.pallas import tpu_sc as plsc

_EPS = 1e-5


def _argmin_loss_body(x_ref, w_ref, idx_ref, loss_ref):
    x = x_ref[0]
    w = w_ref[...]
    T, C = x.shape
    K = w.shape[0]

    s = lax.dot_general(x, w, (((1,), (1,)), ((), ())),
                        preferred_element_type=jnp.float32)
    xs = jnp.sum(x * x, axis=1, keepdims=True)
    w2 = jnp.sum(w * w, axis=1)
    d2 = xs + w2[None, :] - 2.0 * s

    dmin = jnp.min(d2, axis=1, keepdims=True)
    kiota = lax.broadcasted_iota(jnp.int32, (T, K), 1)
    idx = jnp.min(jnp.where(d2 == dmin, kiota, K), axis=1)
    idx_ref[0, 0] = idx

    onehot = kiota == idx[:, None]
    s_win = jnp.sum(jnp.where(onehot, s, 0.0), axis=1)
    w2_win = jnp.sum(jnp.where(onehot, w2[None, :], 0.0), axis=1)

    xs1 = xs[:, 0]
    xn = jnp.maximum(jnp.sqrt(xs1), _EPS)
    wn = jnp.maximum(jnp.sqrt(w2_win), _EPS)
    row = (w2_win / (wn * wn) + xs1 / (xn * xn)
           - 2.0 * s_win / (wn * xn))
    loss_ref[pl.program_id(0)] = jnp.sum(row) / (T * C)


def _argmin_and_loss(x, W):
    B, T, C = x.shape
    K = W.shape[0]
    idx3, loss = pl.pallas_call(
        _argmin_loss_body,
        grid=(B,),
        in_specs=[
            pl.BlockSpec((1, T, C), lambda b: (b, 0, 0)),
            pl.BlockSpec((K, C), lambda b: (0, 0)),
        ],
        out_specs=[
            pl.BlockSpec((1, 1, T), lambda b: (b, 0, 0)),
            pl.BlockSpec(memory_space=pltpu.SMEM),
        ],
        out_shape=[
            jax.ShapeDtypeStruct((B, 1, T), jnp.int32),
            jax.ShapeDtypeStruct((B,), jnp.float32),
        ],
    )(x, W)
    return idx3.reshape(B, T), loss


@functools.cache
def _make_sc_gather(V, D, B):
    info = plsc.get_sparse_core_info()
    NC, NS = info.num_cores, info.num_subcores
    NW = NC * NS
    assert B % (8 * NW) == 0
    b_per_w = B // NW
    mesh = plsc.VectorSubcoreMesh(core_axis_name="c", subcore_axis_name="s")

    @functools.partial(
        pl.kernel, mesh=mesh,
        out_type=jax.ShapeDtypeStruct((B, D), jnp.float32),
        scratch_types=[
            pltpu.VMEM((b_per_w,), jnp.int32),
            pltpu.VMEM((b_per_w, D), jnp.float32),
            pltpu.SemaphoreType.DMA,
        ],
    )
    def gather(table_hbm, idx_hbm, out_hbm, idx_v, rows_v, sem):
        wid = lax.axis_index("s") * NC + lax.axis_index("c")
        base = wid * b_per_w
        pltpu.sync_copy(idx_hbm.at[pl.ds(base, b_per_w)], idx_v)
        pltpu.async_copy(table_hbm.at[idx_v], rows_v, sem).wait()
        pltpu.sync_copy(rows_v, out_hbm.at[pl.ds(base, b_per_w)])

    return gather


def kernel(x, W):
    B, T, C = x.shape
    K = W.shape[0]
    idx, loss = _argmin_and_loss(x, W)
    quant = _make_sc_gather(K, C, B * T)(W, idx.reshape(-1))
    quant = quant.reshape(B, T, C)
    return quant, loss, loss, idx


# speedup vs baseline: 1.0594x; 1.0594x over previous
import functools

import jax
import jax.numpy as jnp
from jax import lax
from jax.experimental import pallas as pl
from jax.experimental.pallas import tpu as pltpu
from jax.experimental.pallas import tpu_sc as plsc

_EPS = 1e-5


def _argmin_loss_body(x_ref, wt_ref, idx_ref, loss_ref):
    x = x_ref[0]
    wt = wt_ref[...]
    T, C = x.shape
    K = wt.shape[1]

    s = lax.dot_general(x, wt, (((1,), (0,)), ((), ())),
                        preferred_element_type=jnp.float32)
    xs = jnp.sum(x * x, axis=1, keepdims=True)
    w2 = jnp.sum(wt * wt, axis=0, keepdims=True)
    d2 = (xs + w2) - 2.0 * s

    dmin = jnp.min(d2, axis=1, keepdims=True)
    eqm = d2 == dmin
    kiota = lax.broadcasted_iota(jnp.int32, (T, K), 1).astype(jnp.float32)
    idx = jnp.min(jnp.where(eqm, kiota, float(K)), axis=1,
                  keepdims=True)
    idx_ref[0] = idx.astype(jnp.int32)

    ef = eqm.astype(jnp.float32)
    w2_win = lax.dot_general(ef, w2, (((1,), (1,)), ((), ())),
                             preferred_element_type=jnp.float32)
    s_win = 0.5 * ((xs + w2_win) - dmin)
    xn = jnp.maximum(jnp.sqrt(xs), _EPS)
    wn = jnp.maximum(jnp.sqrt(w2_win), _EPS)
    row = (w2_win / (wn * wn) + xs / (xn * xn)
           - 2.0 * s_win / (wn * xn))
    loss_ref[pl.program_id(0)] = jnp.sum(row) / (T * C)


def _argmin_and_loss(x, W):
    B, T, C = x.shape
    K = W.shape[0]
    idx3, loss = pl.pallas_call(
        _argmin_loss_body,
        grid=(B,),
        in_specs=[
            pl.BlockSpec((1, T, C), lambda b: (b, 0, 0)),
            pl.BlockSpec((C, K), lambda b: (0, 0)),
        ],
        out_specs=[
            pl.BlockSpec((1, T, 1), lambda b: (b, 0, 0)),
            pl.BlockSpec(memory_space=pltpu.SMEM),
        ],
        out_shape=[
            jax.ShapeDtypeStruct((B, T, 1), jnp.int32),
            jax.ShapeDtypeStruct((B,), jnp.float32),
        ],
    )(x, W.T)
    return idx3.reshape(B, T), loss


@functools.cache
def _make_sc_gather(V, D, B):
    info = plsc.get_sparse_core_info()
    NC, NS = info.num_cores, info.num_subcores
    NW = NC * NS
    assert B % (8 * NW) == 0
    b_per_w = B // NW
    NCHUNK = 4
    assert b_per_w % NCHUNK == 0
    CH = b_per_w // NCHUNK
    assert CH % 8 == 0
    mesh = plsc.VectorSubcoreMesh(core_axis_name="c", subcore_axis_name="s")

    @functools.partial(
        pl.kernel, mesh=mesh,
        out_type=jax.ShapeDtypeStruct((B, D), jnp.float32),
        scratch_types=[
            pltpu.VMEM((b_per_w,), jnp.int32),
            pltpu.VMEM((2, CH, D), jnp.float32),
            pltpu.SemaphoreType.DMA,
        ],
    )
    def gather(table_hbm, idx_hbm, out_hbm, idx_v, rows_v, sem):
        wid = lax.axis_index("s") * NC + lax.axis_index("c")
        base = wid * b_per_w
        pltpu.sync_copy(idx_hbm.at[pl.ds(base, b_per_w)], idx_v)
        h = pltpu.async_copy(table_hbm.at[idx_v.at[pl.ds(0, CH)]],
                             rows_v.at[0], sem)
        for c in range(NCHUNK):
            h.wait()
            if c + 1 < NCHUNK:
                h = pltpu.async_copy(
                    table_hbm.at[idx_v.at[pl.ds((c + 1) * CH, CH)]],
                    rows_v.at[(c + 1) % 2], sem)
            pltpu.sync_copy(rows_v.at[c % 2],
                            out_hbm.at[pl.ds(base + c * CH, CH)])

    return gather


def kernel(x, W):
    B, T, C = x.shape
    K = W.shape[0]
    idx, loss = _argmin_and_loss(x, W)
    quant = _make_sc_gather(K, C, B * T)(W, idx.reshape(-1))
    quant = quant.reshape(B, T, C)
    return quant, loss, loss, idx
